# Initial kernel scaffold; baseline (speedup 1.0000x reference)
import jax
import jax.numpy as jnp
from jax.experimental import pallas as pl


def kernel(z, W):
    raise NotImplementedError("write your pallas kernel here")


# speedup vs baseline: 1.1166x; 1.1166x over previous
import functools

import jax
import jax.numpy as jnp
from jax import lax
from jax.experimental import pallas as pl
from jax.experimental.pallas import tpu as pltpu
from jax.experimental.pallas import tpu_sc as plsc

_CODEBOOK_SIZE = 8192
_LATENT_DIM = 32
_BETA = 0.25

_TOK_BLOCK = 256
_N_TOK = 16384
_N_BLOCKS = _N_TOK // _TOK_BLOCK
_CODE_CHUNK = 4096
_N_CHUNKS = _CODEBOOK_SIZE // _CODE_CHUNK

_INT_MAX = 2**31 - 1


def _argmin_body(z_ref, z2_ref, w_ref, w2_ref, idx_ref, dsum_ref):
    i = pl.program_id(0)
    zb = z_ref[...].astype(jnp.bfloat16)
    z2 = z2_ref[...]

    def chunk(c, carry):
        best_d, best_i = carry
        wc = w_ref[pl.ds(c * _CODE_CHUNK, _CODE_CHUNK), :].astype(jnp.bfloat16)
        w2c = w2_ref[:, pl.ds(c * _CODE_CHUNK, _CODE_CHUNK)]
        m = lax.dot_general(zb, wc, (((1,), (1,)), ((), ())),
                            preferred_element_type=jnp.float32)
        d = (z2 + w2c) - 2.0 * m
        cmin = jnp.min(d, axis=1, keepdims=True)
        iot = lax.broadcasted_iota(jnp.int32, d.shape, 1) + c * _CODE_CHUNK
        cidx = jnp.min(jnp.where(d == cmin, iot, _INT_MAX),
                       axis=1, keepdims=True)
        upd = cmin < best_d
        cmin_r = cmin.astype(jnp.bfloat16).astype(jnp.float32)
        return (jnp.where(upd, cmin_r, best_d), jnp.where(upd, cidx, best_i))

    init = (jnp.full((_TOK_BLOCK, 1), jnp.inf, jnp.float32),
            jnp.zeros((_TOK_BLOCK, 1), jnp.int32))
    best_d, best_i = lax.fori_loop(0, _N_CHUNKS, chunk, init)
    idx_ref[...] = best_i

    @pl.when(i == 0)
    def _():
        dsum_ref[0, 0] = 0.0
    dsum_ref[0, 0] += jnp.sum(best_d)


def _argmin_call(z_flat, z2, W, w2):
    return pl.pallas_call(
        _argmin_body,
        grid=(_N_BLOCKS,),
        in_specs=[
            pl.BlockSpec((_TOK_BLOCK, _LATENT_DIM), lambda i: (i, 0)),
            pl.BlockSpec((_TOK_BLOCK, 1), lambda i: (i, 0)),
            pl.BlockSpec((_CODEBOOK_SIZE, _LATENT_DIM), lambda i: (0, 0)),
            pl.BlockSpec((1, _CODEBOOK_SIZE), lambda i: (0, 0)),
        ],
        out_specs=[
            pl.BlockSpec((_TOK_BLOCK, 1), lambda i: (i, 0)),
            pl.BlockSpec((1, 1), lambda i: (0, 0), memory_space=pltpu.SMEM),
        ],
        out_shape=[
            jax.ShapeDtypeStruct((_N_TOK, 1), jnp.int32),
            jax.ShapeDtypeStruct((1, 1), jnp.float32),
        ],
    )(z_flat, z2, W, w2)


_SC_CORES = 2
_SC_SUBCORES = 16
_NW = _SC_CORES * _SC_SUBCORES
_B_PER_W = _N_TOK // _NW
_IDX_CHUNK = 128
_N_IDX_CHUNKS = _B_PER_W // _IDX_CHUNK


@functools.cache
def _make_sc_gather():
    @functools.partial(
        pl.kernel,
        out_type=jax.ShapeDtypeStruct((_N_TOK, _LATENT_DIM), jnp.float32),
        mesh=plsc.VectorSubcoreMesh(core_axis_name="c", subcore_axis_name="s"),
        scratch_types=[
            pltpu.VMEM((_N_IDX_CHUNKS, _IDX_CHUNK), jnp.int32),
            pltpu.VMEM((_N_IDX_CHUNKS, _IDX_CHUNK, _LATENT_DIM), jnp.float32),
            pltpu.SemaphoreType.DMA,
        ],
        compiler_params=pltpu.CompilerParams(use_tc_tiling_on_sc=False),
    )
    def _sc_gather(w_hbm, idx_hbm, out_hbm, idx_v, rows_v, sem):
        wid = lax.axis_index("s") * _SC_CORES + lax.axis_index("c")
        pltpu.sync_copy(idx_hbm.at[pl.ds(wid * _N_IDX_CHUNKS, _N_IDX_CHUNKS)],
                        idx_v)
        copies = []
        for j in range(_N_IDX_CHUNKS):
            copies.append(
                pltpu.async_copy(w_hbm.at[idx_v.at[j]], rows_v.at[j], sem))
        for c in copies:
            c.wait()
        base = wid * _B_PER_W
        for j in range(_N_IDX_CHUNKS):
            pltpu.sync_copy(rows_v.at[j],
                            out_hbm.at[pl.ds(base + j * _IDX_CHUNK, _IDX_CHUNK)])

    return _sc_gather


def kernel(z, W):
    zp = jnp.transpose(z, (0, 2, 3, 1))
    z_flat = zp.reshape(-1, _LATENT_DIM)
    z2 = jnp.sum(z ** 2, axis=1).reshape(-1, 1)
    w2 = jnp.sum(W ** 2, axis=1).reshape(1, _CODEBOOK_SIZE)
    idx2d, dsum = _argmin_call(z_flat, z2, W, w2)
    indices = idx2d.reshape(-1)
    z_q_flat = _make_sc_gather()(
        W, indices.reshape(_NW * _N_IDX_CHUNKS, _IDX_CHUNK))
    mean_sq = dsum[0, 0] / (_N_TOK * _LATENT_DIM)
    loss = mean_sq + _BETA * mean_sq
    z_q_out = jnp.transpose(z_q_flat.reshape(zp.shape), (0, 3, 1, 2))
    return (z_q_out, indices, loss)


# speedup vs baseline: 1.1316x; 1.0135x over previous
import functools

import jax
import jax.numpy as jnp
from jax import lax
from jax.experimental import pallas as pl
from jax.experimental.pallas import tpu as pltpu
from jax.experimental.pallas import tpu_sc as plsc

_CODEBOOK_SIZE = 8192
_LATENT_DIM = 32
_BETA = 0.25

_TOK_BLOCK = 256
_N_TOK = 16384
_N_BLOCKS = _N_TOK // _TOK_BLOCK
_CODE_CHUNK = 4096
_N_CHUNKS = _CODEBOOK_SIZE // _CODE_CHUNK

_INT_MAX = 2**31 - 1


def _argmin_body(z_ref, z2_ref, w_ref, w2_ref, idx_ref, dsum_ref):
    i = pl.program_id(0)
    zb = z_ref[...]
    z2 = z2_ref[...]

    def chunk(c, carry):
        best_d, best_i = carry
        wc = w_ref[pl.ds(c * _CODE_CHUNK, _CODE_CHUNK), :]
        w2c = w2_ref[:, pl.ds(c * _CODE_CHUNK, _CODE_CHUNK)]
        m = lax.dot_general(zb, wc, (((1,), (1,)), ((), ())),
                            preferred_element_type=jnp.float32)
        d = (z2 + w2c) - 2.0 * m
        cmin = jnp.min(d, axis=1, keepdims=True)
        iot = lax.broadcasted_iota(jnp.int32, d.shape, 1) + c * _CODE_CHUNK
        cidx = jnp.min(jnp.where(d == cmin, iot, _INT_MAX),
                       axis=1, keepdims=True)
        upd = cmin < best_d
        cmin_r = cmin.astype(jnp.bfloat16).astype(jnp.float32)
        return (jnp.where(upd, cmin_r, best_d), jnp.where(upd, cidx, best_i))

    init = (jnp.full((_TOK_BLOCK, 1), jnp.inf, jnp.float32),
            jnp.zeros((_TOK_BLOCK, 1), jnp.int32))
    best_d, best_i = lax.fori_loop(0, _N_CHUNKS, chunk, init)
    idx_ref[...] = best_i

    @pl.when(i == 0)
    def _():
        dsum_ref[0, 0] = 0.0
    dsum_ref[0, 0] += jnp.sum(best_d)


def _argmin_call(z_flat, z2, W, w2):
    return pl.pallas_call(
        _argmin_body,
        grid=(_N_BLOCKS,),
        in_specs=[
            pl.BlockSpec((_TOK_BLOCK, _LATENT_DIM), lambda i: (i, 0)),
            pl.BlockSpec((_TOK_BLOCK, 1), lambda i: (i, 0)),
            pl.BlockSpec((_CODEBOOK_SIZE, _LATENT_DIM), lambda i: (0, 0)),
            pl.BlockSpec((1, _CODEBOOK_SIZE), lambda i: (0, 0)),
        ],
        out_specs=[
            pl.BlockSpec((_TOK_BLOCK, 1), lambda i: (i, 0)),
            pl.BlockSpec((1, 1), lambda i: (0, 0), memory_space=pltpu.SMEM),
        ],
        out_shape=[
            jax.ShapeDtypeStruct((_N_TOK, 1), jnp.int32),
            jax.ShapeDtypeStruct((1, 1), jnp.float32),
        ],
    )(z_flat, z2, W, w2)


_SC_CORES = 2
_SC_SUBCORES = 16
_NW = _SC_CORES * _SC_SUBCORES
_B_PER_W = _N_TOK // _NW
_IDX_CHUNK = 128
_N_IDX_CHUNKS = _B_PER_W // _IDX_CHUNK


@functools.cache
def _make_sc_gather():
    @functools.partial(
        pl.kernel,
        out_type=jax.ShapeDtypeStruct((_N_TOK, _LATENT_DIM), jnp.float32),
        mesh=plsc.VectorSubcoreMesh(core_axis_name="c", subcore_axis_name="s"),
        scratch_types=[
            pltpu.VMEM((_N_IDX_CHUNKS, _IDX_CHUNK), jnp.int32),
            pltpu.VMEM((_N_IDX_CHUNKS, _IDX_CHUNK, _LATENT_DIM), jnp.float32),
            pltpu.SemaphoreType.DMA,
        ],
        compiler_params=pltpu.CompilerParams(use_tc_tiling_on_sc=False),
    )
    def _sc_gather(w_hbm, idx_hbm, out_hbm, idx_v, rows_v, sem):
        wid = lax.axis_index("s") * _SC_CORES + lax.axis_index("c")
        pltpu.sync_copy(idx_hbm.at[pl.ds(wid * _N_IDX_CHUNKS, _N_IDX_CHUNKS)],
                        idx_v)
        copies = []
        for j in range(_N_IDX_CHUNKS):
            copies.append(
                pltpu.async_copy(w_hbm.at[idx_v.at[j]], rows_v.at[j], sem))
        for c in copies:
            c.wait()
        base = wid * _B_PER_W
        for j in range(_N_IDX_CHUNKS):
            pltpu.sync_copy(rows_v.at[j],
                            out_hbm.at[pl.ds(base + j * _IDX_CHUNK, _IDX_CHUNK)])

    return _sc_gather


def kernel(z, W):
    zp = jnp.transpose(z, (0, 2, 3, 1))
    z_flat = zp.reshape(-1, _LATENT_DIM)
    z2 = jnp.sum(z ** 2, axis=1).reshape(-1, 1)
    w2 = jnp.sum(W ** 2, axis=1).reshape(1, _CODEBOOK_SIZE)
    idx2d, dsum = _argmin_call(z_flat.astype(jnp.bfloat16), z2,
                               W.astype(jnp.bfloat16), w2)
    indices = idx2d.reshape(-1)
    z_q_flat = _make_sc_gather()(
        W, indices.reshape(_NW * _N_IDX_CHUNKS, _IDX_CHUNK))
    mean_sq = dsum[0, 0] / (_N_TOK * _LATENT_DIM)
    loss = mean_sq + _BETA * mean_sq
    z_q_out = jnp.transpose(z_q_flat.reshape(zp.shape), (0, 3, 1, 2))
    return (z_q_out, indices, loss)
